# baseline (speedup 1.0000x reference)
import functools

import jax
import jax.numpy as jnp
from jax import lax
from jax.experimental import pallas as pl
from jax.experimental.pallas import tpu as pltpu
from jax.experimental.pallas import tpu_sc as plsc

_NC, _NS, _L = 2, 16, 16


@functools.partial(jax.jit, static_argnames=("B", "P", "Ho", "Wo", "N", "levels"))
def _atlas_sc(u, v, t1, t2, t3, t4, *, B, P, Ho, Wo, N, levels):
    NW = _NC * _NS
    G = B * Ho * Wo * P
    GW = G // NW
    PB = 16 * P
    OPB = PB // P
    NBLK = GW // PB
    NVEC = PB // _L
    CHUNK = 128
    NCH = PB // CHUNK

    mesh = plsc.VectorSubcoreMesh(
        core_axis_name="c", subcore_axis_name="s",
        num_cores=_NC, num_subcores=_NS)

    vmem_i = lambda: pltpu.VMEM((PB,), jnp.int32)
    vmem_r = lambda: pltpu.VMEM((PB, N), jnp.float32)

    @functools.partial(
        pl.kernel,
        out_type=jax.ShapeDtypeStruct((B * Ho * Wo, N), jnp.float32),
        mesh=mesh,
        scratch_types=[
            pltpu.VMEM((PB,), jnp.float32),
            pltpu.VMEM((PB,), jnp.float32),
            [[vmem_i() for _ in range(4)] for _ in range(2)],
            [[vmem_r() for _ in range(4)] for _ in range(2)],
            pltpu.VMEM((OPB, N), jnp.float32),
            pltpu.SemaphoreType.DMA,
        ],
        compiler_params=pltpu.CompilerParams(
            needs_layout_passes=False, use_tc_tiling_on_sc=False),
    )
    def k(u_hbm, v_hbm, x1, x2, x3, x4, out_hbm,
          u_v, v_v, idxs, rows, ob, sem):
        wid = lax.axis_index("c") * _NS + lax.axis_index("s")
        texs = (x1, x2, x3, x4)
        lane = lax.iota(jnp.int32, _L)

        def coords(off, lvl):
            Hl, Wl = levels[lvl]
            u16 = u_v[pl.ds(off, _L)]
            v16 = v_v[pl.ds(off, _L)]
            x = (u16 + 1.0) * (0.5 * (Wl - 1))
            y = (v16 + 1.0) * (0.5 * (Hl - 1))
            xi = jnp.clip(x.astype(jnp.int32), 0, Wl - 2)
            yi = jnp.clip(y.astype(jnp.int32), 0, Hl - 2)
            return x, y, xi, yi

        def cw(lvl):
            s = lvl % 2
            Hl, Wl = levels[lvl]

            def body(i16, c2):
                off = i16 * _L
                x, y, xi, yi = coords(off, lvl)
                p16 = (off + lane) % P
                rbase = (p16 * Hl + yi) * Wl + xi
                idxs[s][0][pl.ds(off, _L)] = rbase
                idxs[s][1][pl.ds(off, _L)] = rbase + 1
                idxs[s][2][pl.ds(off, _L)] = rbase + Wl
                idxs[s][3][pl.ds(off, _L)] = rbase + (Wl + 1)
                return c2

            lax.fori_loop(0, NVEC, body, 0)

        def fire(lvl):
            s = lvl % 2
            cps = []
            for t in range(4):
                for c in range(NCH):
                    cps.append(pltpu.async_copy(
                        texs[lvl].at[idxs[s][t].at[pl.ds(c * CHUNK, CHUNK)]],
                        rows[s][t].at[pl.ds(c * CHUNK, CHUNK), :], sem))
            return cps

        def combine(lvl):
            s = lvl % 2

            def body(i16, c2):
                off = i16 * _L
                x, y, xi, yi = coords(off, lvl)
                fx = x - xi.astype(jnp.float32)
                fy = y - yi.astype(jnp.float32)
                gx = 1.0 - fx
                gy = 1.0 - fy
                wts = (gy * gx, gy * fx, fy * gx, fy * fx)
                rvec = off + lane
                pix = rvec // P
                for w in range(N):
                    cw_ = jnp.full((_L,), w, jnp.int32)
                    acc = None
                    for t in range(4):
                        val = plsc.load_gather(rows[s][t], [rvec, cw_])
                        contrib = wts[t] * val
                        acc = contrib if acc is None else acc + contrib
                    plsc.addupdate_scatter(ob, [pix, cw_], acc)
                return c2

            lax.fori_loop(0, NVEC, body, 0)

        def block(blk, carry):
            base = pl.multiple_of(wid * GW + blk * PB, PB)
            pltpu.sync_copy(u_hbm.at[pl.ds(base, PB)], u_v)
            pltpu.sync_copy(v_hbm.at[pl.ds(base, PB)], v_v)
            for op in range(OPB):
                ob[op] = jnp.zeros((N,), jnp.float32)
            cw(0)
            cps = fire(0)
            cw(1)
            for lvl in range(4):
                for cp in cps:
                    cp.wait()
                if lvl < 3:
                    cps = fire(lvl + 1)
                if lvl < 2:
                    cw(lvl + 2)
                combine(lvl)
            pltpu.sync_copy(
                ob, out_hbm.at[pl.ds(pl.multiple_of(base // P, OPB), OPB), :])
            return carry

        lax.fori_loop(0, NBLK, block, 0)

    return k(u, v, t1, t2, t3, t4)


def kernel(iuv, layer1, layer2, layer3, layer4):
    B, P, Ho, Wo, _ = iuv.shape
    N = layer1.shape[1]
    layers = (layer1, layer2, layer3, layer4)
    levels = tuple((l.shape[2], l.shape[3]) for l in layers)
    g = jnp.transpose(iuv, (0, 2, 3, 1, 4))
    u = g[..., 0].reshape(-1)
    v = g[..., 1].reshape(-1)
    texs = [jnp.transpose(l, (0, 2, 3, 1)).reshape(-1, N) for l in layers]
    out = _atlas_sc(u, v, *texs, B=B, P=P, Ho=Ho, Wo=Wo, N=N, levels=levels)
    return out.reshape(B, Ho, Wo, N).transpose(0, 3, 1, 2)


# speedup vs baseline: 2.7817x; 2.7817x over previous
import functools

import jax
import jax.numpy as jnp
from jax import lax
from jax.experimental import pallas as pl
from jax.experimental.pallas import tpu as pltpu
from jax.experimental.pallas import tpu_sc as plsc

_NC, _NS, _L = 2, 16, 16


@functools.partial(jax.jit, static_argnames=("B", "P", "Ho", "Wo", "N", "levels"))
def _atlas_sc(u, v, t1, t2, t3, t4, *, B, P, Ho, Wo, N, levels):
    NW = _NC * _NS
    G = B * Ho * Wo * P
    GW = G // NW
    PB = 16 * P
    OPB = PB // P
    NBLK = GW // PB
    NVEC = PB // _L
    CHUNK = 128
    NCH = PB // CHUNK

    mesh = plsc.VectorSubcoreMesh(
        core_axis_name="c", subcore_axis_name="s",
        num_cores=_NC, num_subcores=_NS)

    vmem_i = lambda: pltpu.VMEM((PB,), jnp.int32)
    vmem_r = lambda: pltpu.VMEM((PB, N), jnp.float32)

    @functools.partial(
        pl.kernel,
        out_type=jax.ShapeDtypeStruct((B * Ho * Wo, N), jnp.float32),
        mesh=mesh,
        scratch_types=[
            pltpu.VMEM((PB,), jnp.float32),
            pltpu.VMEM((PB,), jnp.float32),
            [[vmem_i() for _ in range(4)] for _ in range(2)],
            [[vmem_r() for _ in range(4)] for _ in range(2)],
            [[pltpu.VMEM((PB,), jnp.float32) for _ in range(2)]
             for _ in range(2)],
            pltpu.VMEM((OPB, N), jnp.float32),
            pltpu.SemaphoreType.DMA,
        ],
        compiler_params=pltpu.CompilerParams(
            needs_layout_passes=False, use_tc_tiling_on_sc=False),
    )
    def k(u_hbm, v_hbm, x1, x2, x3, x4, out_hbm,
          u_v, v_v, idxs, rows, f_v, ob, sem):
        wid = lax.axis_index("c") * _NS + lax.axis_index("s")
        texs = (x1, x2, x3, x4)
        lane = lax.iota(jnp.int32, _L)

        def cw(lvl):
            s = lvl % 2
            Hl, Wl = levels[lvl]

            def body(i16, c2):
                off = i16 * _L
                u16 = u_v[pl.ds(off, _L)]
                v16 = v_v[pl.ds(off, _L)]
                x = (u16 + 1.0) * (0.5 * (Wl - 1))
                y = (v16 + 1.0) * (0.5 * (Hl - 1))
                xi = jnp.clip(x.astype(jnp.int32), 0, Wl - 2)
                yi = jnp.clip(y.astype(jnp.int32), 0, Hl - 2)
                p16 = (off + lane) % P
                rbase = (p16 * Hl + yi) * Wl + xi
                idxs[s][0][pl.ds(off, _L)] = rbase
                idxs[s][1][pl.ds(off, _L)] = rbase + 1
                idxs[s][2][pl.ds(off, _L)] = rbase + Wl
                idxs[s][3][pl.ds(off, _L)] = rbase + (Wl + 1)
                f_v[s][0][pl.ds(off, _L)] = x - xi.astype(jnp.float32)
                f_v[s][1][pl.ds(off, _L)] = y - yi.astype(jnp.float32)
                return c2

            lax.fori_loop(0, NVEC, body, 0)

        def fire(lvl):
            s = lvl % 2
            cps = []
            for t in range(4):
                for c in range(NCH):
                    cps.append(pltpu.async_copy(
                        texs[lvl].at[idxs[s][t].at[pl.ds(c * CHUNK, CHUNK)]],
                        rows[s][t].at[pl.ds(c * CHUNK, CHUNK), :], sem))
            return cps

        def combine(lvl):
            s = lvl % 2

            def body(op, c2):
                acc = jnp.zeros((N,), jnp.float32)
                for j in range(P):
                    i = op * P + j
                    ii = jnp.full((_L,), i, jnp.int32)
                    fx = plsc.load_gather(f_v[s][0], [ii])
                    fy = plsc.load_gather(f_v[s][1], [ii])
                    gx = 1.0 - fx
                    gy = 1.0 - fy
                    acc = (acc
                           + (gy * gx) * rows[s][0][i]
                           + (gy * fx) * rows[s][1][i]
                           + (fy * gx) * rows[s][2][i]
                           + (fy * fx) * rows[s][3][i])
                if lvl == 0:
                    ob[op] = acc
                else:
                    ob[op] = ob[op] + acc
                return c2

            lax.fori_loop(0, OPB, body, 0)

        def block(blk, carry):
            base = pl.multiple_of(wid * GW + blk * PB, PB)
            pltpu.sync_copy(u_hbm.at[pl.ds(base, PB)], u_v)
            pltpu.sync_copy(v_hbm.at[pl.ds(base, PB)], v_v)
            cw(0)
            cps = fire(0)
            for lvl in range(4):
                if lvl < 3:
                    cw(lvl + 1)
                for cp in cps:
                    cp.wait()
                if lvl < 3:
                    cps = fire(lvl + 1)
                combine(lvl)
            pltpu.sync_copy(
                ob, out_hbm.at[pl.ds(pl.multiple_of(base // P, OPB), OPB), :])
            return carry

        lax.fori_loop(0, NBLK, block, 0)

    return k(u, v, t1, t2, t3, t4)


def kernel(iuv, layer1, layer2, layer3, layer4):
    B, P, Ho, Wo, _ = iuv.shape
    N = layer1.shape[1]
    layers = (layer1, layer2, layer3, layer4)
    levels = tuple((l.shape[2], l.shape[3]) for l in layers)
    g = jnp.transpose(iuv, (0, 2, 3, 1, 4))
    u = g[..., 0].reshape(-1)
    v = g[..., 1].reshape(-1)
    texs = [jnp.transpose(l, (0, 2, 3, 1)).reshape(-1, N) for l in layers]
    out = _atlas_sc(u, v, *texs, B=B, P=P, Ho=Ho, Wo=Wo, N=N, levels=levels)
    return out.reshape(B, Ho, Wo, N).transpose(0, 3, 1, 2)
